# baseline (speedup 1.0000x reference)
import jax
import jax.numpy as jnp
from jax.experimental import pallas as pl
from jax.experimental.pallas import tpu as pltpu

_B, _Q, _C = 64, 900, 256
_QP = 1024
_K = 100
_KP = 128

_NEG = float("-inf")
_BIG = 1 << 30


def _topk_kernel(logits_ref, boxes_ref, ts_ref,
                 scores_ref, labels_ref, oboxes_ref,
                 p_ref):
    x = logits_ref[0]
    m_row = jnp.max(x, axis=1, keepdims=True)
    e = jnp.exp(x - m_row)
    s_row = jnp.sum(e, axis=1, keepdims=True)
    p = e / s_row
    row_id2 = jax.lax.broadcasted_iota(jnp.int32, (_QP, _C), 0)
    p = jnp.where(row_id2 < _Q, p, _NEG)
    p_ref[:] = p

    rm0 = jnp.max(p.reshape(8, 128, _C), axis=2)
    flat_rid = jax.lax.broadcasted_iota(jnp.int32, (8, 128), 0) * 128 + \
        jax.lax.broadcasted_iota(jnp.int32, (8, 128), 1)
    flat_pos = flat_rid
    col_id = jax.lax.broadcasted_iota(jnp.int32, (1, _C), 1)
    sub_id = jax.lax.broadcasted_iota(jnp.int32, (_KP, 1), 0)

    def body(i, carry):
        rm, sc_acc, lb_acc, rows_col = carry
        m = jnp.max(rm, keepdims=True)
        r = jnp.min(jnp.where(rm == m, flat_rid, _BIG))
        row = p_ref[pl.ds(r, 1), :]
        c = jnp.min(jnp.where(row == m, col_id, _BIG), keepdims=True)
        masked = jnp.where(col_id == c, _NEG, row)
        p_ref[pl.ds(r, 1), :] = masked
        rowm2 = jnp.max(masked, keepdims=True)
        slot = flat_pos == i
        return (jnp.where(flat_rid == r, rowm2, rm),
                jnp.where(slot, m, sc_acc),
                jnp.where(slot, c, lb_acc),
                jnp.where(sub_id == i, r, rows_col))

    rm, sc_acc, lb_acc, rows_col = jax.lax.fori_loop(
        0, _K, body,
        (rm0,
         jnp.zeros((8, 128), jnp.float32),
         jnp.zeros((8, 128), jnp.int32),
         jnp.zeros((_KP, 1), jnp.int32)))

    scores_ref[0] = sc_acc
    labels_ref[0] = lb_acc

    onehot = (rows_col == jax.lax.broadcasted_iota(
        jnp.int32, (1, _QP), 1)).astype(jnp.float32)
    picked = jax.lax.dot_general(
        onehot, boxes_ref[0], (((1,), (0,)), ((), ())),
        preferred_element_type=jnp.float32)
    xc, yc = picked[:, 0:1], picked[:, 1:2]
    w2, h2 = picked[:, 2:3] * 0.5, picked[:, 3:4] * 0.5
    xyxy = jnp.concatenate([xc - w2, yc - h2, xc + w2, yc + h2], axis=1)
    img_h = ts_ref[0, 0, 0]
    img_w = ts_ref[0, 0, 1]
    lane4 = jax.lax.broadcasted_iota(jnp.int32, (1, 4), 1)
    scale = jnp.where(lane4 % 2 == 0, img_w, img_h)
    oboxes_ref[0] = xyxy * scale


@jax.jit
def kernel(pred_logits, pred_boxes, target_sizes):
    xp = jnp.pad(pred_logits, ((0, 0), (0, _QP - _Q), (0, 0)))
    bp = jnp.pad(pred_boxes, ((0, 0), (0, _QP - _Q), (0, 0)))
    ts = target_sizes.astype(jnp.float32).reshape(_B, 1, 2)

    scores, labels, boxes = pl.pallas_call(
        _topk_kernel,
        grid=(_B,),
        in_specs=[
            pl.BlockSpec((1, _QP, _C), lambda b: (b, 0, 0)),
            pl.BlockSpec((1, _QP, 4), lambda b: (b, 0, 0)),
            pl.BlockSpec((1, 1, 2), lambda b: (b, 0, 0)),
        ],
        out_specs=[
            pl.BlockSpec((1, 8, 128), lambda b: (b, 0, 0)),
            pl.BlockSpec((1, 8, 128), lambda b: (b, 0, 0)),
            pl.BlockSpec((1, _KP, 4), lambda b: (b, 0, 0)),
        ],
        out_shape=[
            jax.ShapeDtypeStruct((_B, 8, 128), jnp.float32),
            jax.ShapeDtypeStruct((_B, 8, 128), jnp.int32),
            jax.ShapeDtypeStruct((_B, _KP, 4), jnp.float32),
        ],
        scratch_shapes=[
            pltpu.VMEM((_QP, _C), jnp.float32),
        ],
        compiler_params=pltpu.CompilerParams(
            dimension_semantics=("parallel",),
        ),
    )(xp, bp, ts)

    return (scores.reshape(_B, 1024)[:, :_K],
            labels.reshape(_B, 1024)[:, :_K],
            boxes[:, :_K, :])


# speedup vs baseline: 1.1618x; 1.1618x over previous
import jax
import jax.numpy as jnp
from jax.experimental import pallas as pl
from jax.experimental.pallas import tpu as pltpu

_B, _Q, _C = 64, 900, 256
_QP = 1024
_K = 100
_KP = 128
_BB = 8

_NEG = float("-inf")
_BIG = 1 << 30


def _topk_kernel(logits_ref, boxes_ref, ts_ref,
                 scores_ref, labels_ref, oboxes_ref,
                 p_ref):
    row_id2 = jax.lax.broadcasted_iota(jnp.int32, (_QP, _C), 0)
    flat_rid = jax.lax.broadcasted_iota(jnp.int32, (8, 128), 0) * 128 + \
        jax.lax.broadcasted_iota(jnp.int32, (8, 128), 1)
    col_id = jax.lax.broadcasted_iota(jnp.int32, (1, _C), 1)
    col_id8 = jax.lax.broadcasted_iota(jnp.int32, (8, _C), 1)
    sub_id8 = jax.lax.broadcasted_iota(jnp.int32, (8, 1), 0)
    sub_id_kp = jax.lax.broadcasted_iota(jnp.int32, (_KP, 1), 0)

    rm0 = []
    for b in range(_BB):
        x = logits_ref[b]
        m_row = jnp.max(x, axis=1, keepdims=True)
        e = jnp.exp(x - m_row)
        s_row = jnp.sum(e, axis=1, keepdims=True)
        p = e / s_row
        p = jnp.where(row_id2 < _Q, p, _NEG)
        p_ref[b] = p
        rm0.append(jnp.max(p.reshape(8, 128, _C), axis=2))

    zf = jnp.zeros((8, 128), jnp.float32)
    zi = jnp.zeros((8, 128), jnp.int32)
    zr = jnp.zeros((_KP, 1), jnp.int32)
    carry0 = (tuple(rm0), (zf,) * _BB, (zi,) * _BB, (zr,) * _BB)

    def body(i, carry):
        rms, scs, lbs, rws = carry
        out = ([], [], [], [])
        for b in range(_BB):
            rm = rms[b]
            m = jnp.max(rm, keepdims=True)
            r = jnp.min(jnp.where(rm == m, flat_rid, _BIG))
            r8 = (r // 8) * 8
            blk = p_ref[b, pl.ds(r8, 8), :]
            rowmask = sub_id8 == (r - r8)
            c = jnp.min(jnp.where(rowmask & (blk == m), col_id8, _BIG),
                        keepdims=True)
            masked = jnp.where(rowmask & (col_id8 == c), _NEG, blk)
            p_ref[b, pl.ds(r8, 8), :] = masked
            rowm2 = jnp.max(jnp.where(rowmask, masked, _NEG), keepdims=True)
            slot = flat_rid == i
            out[0].append(jnp.where(flat_rid == r, rowm2, rm))
            out[1].append(jnp.where(slot, m, scs[b]))
            out[2].append(jnp.where(slot, c, lbs[b]))
            out[3].append(jnp.where(sub_id_kp == i, r, rws[b]))
        return tuple(tuple(o) for o in out)

    _, scs, lbs, rws = jax.lax.fori_loop(0, _K, body, carry0)

    lane4 = jax.lax.broadcasted_iota(jnp.int32, (1, 4), 1)
    qp_id = jax.lax.broadcasted_iota(jnp.int32, (1, _QP), 1)
    for b in range(_BB):
        img_h = ts_ref[b, 0, 0]
        img_w = ts_ref[b, 0, 1]
        scale = jnp.where(lane4 % 2 == 0, img_w, img_h)
        scores_ref[b] = scs[b]
        labels_ref[b] = lbs[b]
        onehot = (rws[b] == qp_id).astype(jnp.float32)
        picked = jax.lax.dot_general(
            onehot, boxes_ref[b], (((1,), (0,)), ((), ())),
            preferred_element_type=jnp.float32,
            precision=jax.lax.Precision.HIGHEST)
        xc, yc = picked[:, 0:1], picked[:, 1:2]
        w2, h2 = picked[:, 2:3] * 0.5, picked[:, 3:4] * 0.5
        xyxy = jnp.concatenate([xc - w2, yc - h2, xc + w2, yc + h2], axis=1)
        oboxes_ref[b] = xyxy * scale


@jax.jit
def kernel(pred_logits, pred_boxes, target_sizes):
    xp = jnp.pad(pred_logits, ((0, 0), (0, _QP - _Q), (0, 0)))
    bp = jnp.pad(pred_boxes, ((0, 0), (0, _QP - _Q), (0, 0)))
    ts = target_sizes.astype(jnp.float32).reshape(_B, 1, 2)

    ng = _B // _BB
    scores, labels, boxes = pl.pallas_call(
        _topk_kernel,
        grid=(ng,),
        in_specs=[
            pl.BlockSpec((_BB, _QP, _C), lambda g: (g, 0, 0)),
            pl.BlockSpec((_BB, _QP, 4), lambda g: (g, 0, 0)),
            pl.BlockSpec((_BB, 1, 2), lambda g: (g, 0, 0)),
        ],
        out_specs=[
            pl.BlockSpec((_BB, 8, 128), lambda g: (g, 0, 0)),
            pl.BlockSpec((_BB, 8, 128), lambda g: (g, 0, 0)),
            pl.BlockSpec((_BB, _KP, 4), lambda g: (g, 0, 0)),
        ],
        out_shape=[
            jax.ShapeDtypeStruct((_B, 8, 128), jnp.float32),
            jax.ShapeDtypeStruct((_B, 8, 128), jnp.int32),
            jax.ShapeDtypeStruct((_B, _KP, 4), jnp.float32),
        ],
        scratch_shapes=[
            pltpu.VMEM((_BB, _QP, _C), jnp.float32),
        ],
        compiler_params=pltpu.CompilerParams(
            dimension_semantics=("parallel",),
        ),
    )(xp, bp, ts)

    return (scores.reshape(_B, 1024)[:, :_K],
            labels.reshape(_B, 1024)[:, :_K],
            boxes[:, :_K, :])


# speedup vs baseline: 1.1989x; 1.0319x over previous
import jax
import jax.numpy as jnp
from jax.experimental import pallas as pl
from jax.experimental.pallas import tpu as pltpu

_B, _Q, _C = 64, 900, 256
_QP = 1024
_K = 100
_KP = 128
_BB = 16

_NEG = float("-inf")
_BIG = 1 << 30


def _topk_kernel(logits_ref, boxes_ref, ts_ref,
                 scores_ref, labels_ref, oboxes_ref,
                 p_ref):
    row_id2 = jax.lax.broadcasted_iota(jnp.int32, (_QP, _C), 0)
    flat_rid = jax.lax.broadcasted_iota(jnp.int32, (8, 128), 0) * 128 + \
        jax.lax.broadcasted_iota(jnp.int32, (8, 128), 1)
    col_id8 = jax.lax.broadcasted_iota(jnp.int32, (8, _C), 1)
    sub_id8 = jax.lax.broadcasted_iota(jnp.int32, (8, 1), 0)
    lane_kp = jax.lax.broadcasted_iota(jnp.int32, (1, _KP), 1)

    rm0 = []
    for b in range(_BB):
        x = logits_ref[b]
        m_row = jnp.max(x, axis=1, keepdims=True)
        e = jnp.exp(x - m_row)
        s_row = jnp.sum(e, axis=1, keepdims=True)
        p = e / s_row
        p = jnp.where(row_id2 < _Q, p, _NEG)
        p_ref[b] = p
        rm0.append(jnp.max(p.reshape(8, 128, _C), axis=2))

    zf = jnp.zeros((8, 128), jnp.float32)
    zi = jnp.zeros((8, 128), jnp.int32)
    zr = jnp.zeros((1, _KP), jnp.int32)
    carry0 = (tuple(rm0), (zf,) * _BB, (zi,) * _BB, (zr,) * _BB)

    def body(i, carry):
        rms, scs, lbs, rws = carry
        out = ([], [], [], [])
        for b in range(_BB):
            rm = rms[b]
            m = jnp.max(rm, keepdims=True)
            r = jnp.min(jnp.where(rm == m, flat_rid, _BIG))
            r8 = (r // 8) * 8
            blk = p_ref[b, pl.ds(r8, 8), :]
            rowmask = sub_id8 == (r - r8)
            c = jnp.min(jnp.where(rowmask & (blk == m), col_id8, _BIG),
                        keepdims=True)
            masked = jnp.where(rowmask & (col_id8 == c), _NEG, blk)
            p_ref[b, pl.ds(r8, 8), :] = masked
            rowm2 = jnp.max(jnp.where(rowmask, masked, _NEG), keepdims=True)
            slot = flat_rid == i
            out[0].append(jnp.where(flat_rid == r, rowm2, rm))
            out[1].append(jnp.where(slot, m, scs[b]))
            out[2].append(jnp.where(slot, c, lbs[b]))
            out[3].append(jnp.where(lane_kp == i, r, rws[b]))
        return tuple(tuple(o) for o in out)

    _, scs, lbs, rws = jax.lax.fori_loop(0, _K, body, carry0)

    sub4 = jax.lax.broadcasted_iota(jnp.int32, (4, 1), 0)
    qp_sub = jax.lax.broadcasted_iota(jnp.int32, (_QP, 1), 0)
    for b in range(_BB):
        img_h = ts_ref[b, 0, 0]
        img_w = ts_ref[b, 0, 1]
        scale = jnp.where(sub4 % 2 == 0, img_w, img_h)
        scores_ref[b] = scs[b]
        labels_ref[b] = lbs[b]
        onehot = (qp_sub == rws[b]).astype(jnp.float32)
        pt = jax.lax.dot_general(
            boxes_ref[b], onehot, (((1,), (0,)), ((), ())),
            preferred_element_type=jnp.float32,
            precision=jax.lax.Precision.HIGHEST)
        w2, h2 = pt[2:3, :] * 0.5, pt[3:4, :] * 0.5
        xyxy = jnp.concatenate(
            [pt[0:1, :] - w2, pt[1:2, :] - h2,
             pt[0:1, :] + w2, pt[1:2, :] + h2], axis=0)
        oboxes_ref[b] = xyxy * scale


@jax.jit
def kernel(pred_logits, pred_boxes, target_sizes):
    xp = jnp.pad(pred_logits, ((0, 0), (0, _QP - _Q), (0, 0)))
    bp = jnp.pad(pred_boxes, ((0, 0), (0, _QP - _Q), (0, 0))).transpose(0, 2, 1)
    ts = target_sizes.astype(jnp.float32).reshape(_B, 1, 2)

    ng = _B // _BB
    scores, labels, boxes = pl.pallas_call(
        _topk_kernel,
        grid=(ng,),
        in_specs=[
            pl.BlockSpec((_BB, _QP, _C), lambda g: (g, 0, 0)),
            pl.BlockSpec((_BB, 4, _QP), lambda g: (g, 0, 0)),
            pl.BlockSpec((_BB, 1, 2), lambda g: (g, 0, 0)),
        ],
        out_specs=[
            pl.BlockSpec((_BB, 8, 128), lambda g: (g, 0, 0)),
            pl.BlockSpec((_BB, 8, 128), lambda g: (g, 0, 0)),
            pl.BlockSpec((_BB, 4, _KP), lambda g: (g, 0, 0)),
        ],
        out_shape=[
            jax.ShapeDtypeStruct((_B, 8, 128), jnp.float32),
            jax.ShapeDtypeStruct((_B, 8, 128), jnp.int32),
            jax.ShapeDtypeStruct((_B, 4, _KP), jnp.float32),
        ],
        scratch_shapes=[
            pltpu.VMEM((_BB, _QP, _C), jnp.float32),
        ],
        compiler_params=pltpu.CompilerParams(
            dimension_semantics=("parallel",),
        ),
    )(xp, bp, ts)

    return (scores.reshape(_B, 1024)[:, :_K],
            labels.reshape(_B, 1024)[:, :_K],
            boxes.transpose(0, 2, 1)[:, :_K, :])
